# baseline (speedup 1.0000x reference)
import functools

import jax
import jax.numpy as jnp
from jax import lax
from jax.experimental import pallas as pl
from jax.experimental.pallas import tpu as pltpu
from jax.experimental.pallas import tpu_sc as plsc

NBX = 512
NBY = 512
NMOV = 1000000
BSX = 1.0 / NBX
BSY = 1.0 / NBY
K = 4
KW = K * K // 2

NC = 2
NS = 16
NW = NC * NS
L = 16

CHUNK = 1600
NCHUNK = NMOV // CHUNK
ITERS = 24

ROWS_PER_TILE = NBX // NS
BSTAGE = 40


def _mesh():
    return plsc.VectorSubcoreMesh(
        core_axis_name="c", subcore_axis_name="s",
        num_cores=NC, num_subcores=NS)


def _params():
    return pltpu.CompilerParams(
        needs_layout_passes=False, use_tc_tiling_on_sc=False)


def _splat_i32(x):
    return jnp.full((L,), 0, jnp.int32) + x


def _pack_pair(a, b):
    ai = plsc.bitcast(a, jnp.int32)
    bi = plsc.bitcast(b, jnp.int32)
    return jnp.bitwise_or(
        lax.shift_right_logical(ai, 16),
        jnp.bitwise_and(bi, jnp.int32(-65536)))


def _unpack_pair(w):
    lo = plsc.bitcast(lax.shift_left(w, 16), jnp.float32)
    hi = plsc.bitcast(w, jnp.float32)
    return lo, hi


def _fused_body(pos_hbm, nsx_hbm, nsy_hbm, map_hbm, area_hbm, table_hbm,
                rowbuf, obuf, xv, yv, sxv, syv, idxv, lxv, lyv, patches,
                areav, bsem0, bsem1, gsem0, gsem1, isem, osem0, osem1):
    c = lax.axis_index("c")
    sid = lax.axis_index("s")
    wid = sid * NC + c
    iota = lax.broadcasted_iota(jnp.int32, (L,), 0)
    bsems = (bsem0, bsem1)
    gsems = (gsem0, gsem1)
    osems = (osem0, osem1)

    r0 = sid * ROWS_PER_TILE
    base = jnp.minimum(r0, NBX - BSTAGE)
    pltpu.sync_copy(map_hbm.at[pl.ds(base, BSTAGE)], rowbuf)
    tb = c * (NBX * NBY)

    def build_row(r, pr, t):
        @pl.when(t >= 1)
        def _():
            pltpu.make_async_copy(
                obuf.at[pr], table_hbm.at[pl.ds(tb + (r - 2) * NBY, NBY)],
                bsems[pr]).wait()

        ob = obuf.at[pr]
        rlocs = [jnp.minimum(r + kx, NBX - 1) - base for kx in range(K)]

        @plsc.parallel_loop(0, NBY // L - 1, unroll=2)
        def per_cb(cb):
            ci = cb * L + iota
            for kx in range(K):
                vals = [rowbuf[rlocs[kx], pl.ds(cb * L + ky, L)]
                        for ky in range(K)]
                for w in range(2):
                    wv = _pack_pair(vals[2 * w], vals[2 * w + 1])
                    plsc.store_scatter(ob, [ci, _splat_i32(kx * 2 + w)], wv)

        cl = (NBY // L - 1) * L + iota
        for kx in range(K):
            rv = _splat_i32(rlocs[kx])
            vals = []
            for ky in range(K):
                cv = jnp.minimum(cl + ky, NBY - 1)
                vals.append(plsc.load_gather(rowbuf, [rv, cv]))
            for w in range(2):
                wv = _pack_pair(vals[2 * w], vals[2 * w + 1])
                plsc.store_scatter(ob, [cl, _splat_i32(kx * 2 + w)], wv)

        pltpu.async_copy(
            ob, table_hbm.at[pl.ds(tb + r * NBY, NBY)], bsems[pr])

    def build_trip(t, carry):
        build_row(r0 + 2 * t, 0, t)
        build_row(r0 + 2 * t + 1, 1, t)
        return carry

    lax.fori_loop(0, ROWS_PER_TILE // 2, build_trip, 0)

    for rl in (ROWS_PER_TILE - 2, ROWS_PER_TILE - 1):
        pltpu.make_async_copy(
            obuf.at[rl % 2], table_hbm.at[pl.ds(tb + (r0 + rl) * NBY, NBY)],
            bsems[rl % 2]).wait()

    plsc.subcore_barrier()

    def gather_descs(p, make):
        return [make(table_hbm.at[idxv.at[p]], patches.at[p], gsems[p])]

    def input_descs(cid, q, make):
        off = cid * CHUNK
        return [
            make(pos_hbm.at[pl.ds(off, CHUNK)], xv.at[q], isem),
            make(pos_hbm.at[pl.ds(NMOV + off, CHUNK)], yv.at[q], isem),
            make(nsx_hbm.at[pl.ds(off, CHUNK)], sxv.at[q], isem),
            make(nsy_hbm.at[pl.ds(off, CHUNK)], syv.at[q], isem),
        ]

    def fire_inputs(cid, q):
        @pl.when(cid < NCHUNK)
        def _():
            input_descs(cid, q, pltpu.async_copy)

    def prep(cid, p, q):
        @pl.when(cid < NCHUNK)
        def _():
            for d in input_descs(cid, q, pltpu.make_async_copy):
                d.wait()

            @plsc.parallel_loop(0, CHUNK // L, unroll=4)
            def idx_pass(n0):
                s = pl.ds(n0 * L, L)
                ix = (xv[q, s] * float(NBX)).astype(jnp.int32)
                iy = (yv[q, s] * float(NBY)).astype(jnp.int32)
                idxv[p, s] = ix * NBY + iy + tb
                lxv[p, s] = ix.astype(jnp.float32) * BSX + BSX
                lyv[p, s] = iy.astype(jnp.float32) * BSY + BSY

            gather_descs(p, pltpu.async_copy)

    fire_inputs(wid, 0)
    fire_inputs(wid + NW, 1)
    prep(wid, 0, 0)

    def process(i, p, q, j6, drain_always):
        cid = wid + NW * i
        fire_inputs(wid + NW * (i + 2), (q + 2) % 3)
        prep(wid + NW * (i + 1), 1 - p, (q + 1) % 3)

        @pl.when(cid < NCHUNK)
        def _():
            for d in gather_descs(p, pltpu.make_async_copy):
                d.wait()

            def drain():
                pltpu.make_async_copy(
                    areav.at[p], area_hbm.at[pl.ds(cid * CHUNK, CHUNK)],
                    osems[p]).wait()

            if drain_always:
                drain()
            else:
                pl.when(j6 >= 1)(drain)

            up = patches.at[p]

            @plsc.parallel_loop(0, CHUNK // L, unroll=2)
            def red_pass(n0):
                s = pl.ds(n0 * L, L)
                x = xv[q, s]
                y = yv[q, s]
                xmax = jnp.minimum(x + sxv[q, s], 1.0)
                ymax = jnp.minimum(y + syv[q, s], 1.0)
                lx1 = lxv[p, s]
                ly1 = lyv[p, s]
                ax = xmax - lx1
                ay = ymax - ly1
                ovx = [jnp.minimum(xmax, lx1) - x,
                       jnp.maximum(jnp.minimum(ax, BSX), 0.0),
                       jnp.maximum(jnp.minimum(ax - BSX, BSX), 0.0),
                       jnp.maximum(ax - 2 * BSX, 0.0)]
                ovy = [jnp.minimum(ymax, ly1) - y,
                       jnp.maximum(jnp.minimum(ay, BSY), 0.0),
                       jnp.maximum(jnp.minimum(ay - BSY, BSY), 0.0),
                       jnp.maximum(ay - 2 * BSY, 0.0)]
                rowi = _splat_i32(n0 * L) + iota
                pk01 = plsc.pack(ovy[0], ovy[1],
                                 format=plsc.PackFormat.INTERLEAVED)
                pk23 = plsc.pack(ovy[2], ovy[3],
                                 format=plsc.PackFormat.INTERLEAVED)
                accb = jnp.zeros((2 * L,), jnp.bfloat16)
                for kx in range(K):
                    w0 = plsc.load_gather(up, [rowi, _splat_i32(kx * 2)])
                    w1 = plsc.load_gather(up, [rowi, _splat_i32(kx * 2 + 1)])
                    oxp = plsc.pack(ovx[kx], ovx[kx],
                                    format=plsc.PackFormat.INTERLEAVED)
                    b0 = plsc.bitcast(w0, jnp.bfloat16)
                    b1 = plsc.bitcast(w1, jnp.bfloat16)
                    accb = accb + (b0 * (pk01 * oxp) + b1 * (pk23 * oxp))
                ae, ao = plsc.unpack(accb,
                                     format=plsc.PackFormat.INTERLEAVED)
                areav[p, s] = ae + ao

            pltpu.async_copy(
                areav.at[p], area_hbm.at[pl.ds(cid * CHUNK, CHUNK)],
                osems[p])

    def per_six(j6, carry):
        for k in range(6):
            process(6 * j6 + k, k % 2, k % 3, j6, k >= 2)
        return carry

    lax.fori_loop(0, ITERS // 6, per_six, 0)

    nv = lax.div(NCHUNK - wid + NW - 1, NW)
    for p in range(2):
        i_p = jnp.where(lax.rem(nv - 1, 2) == p, nv - 1, nv - 2)
        lastc = wid + NW * i_p

        @pl.when(i_p >= 0)
        def _(lastc=lastc, p=p):
            pltpu.make_async_copy(
                areav.at[p], area_hbm.at[pl.ds(lastc * CHUNK, CHUNK)],
                osems[p]).wait()


def kernel(pos, node_size_x, node_size_y, utilization_map):
    f = functools.partial(
        pl.kernel,
        out_type=(
            jax.ShapeDtypeStruct((NMOV,), jnp.float32),
            jax.ShapeDtypeStruct((NC * NBX * NBY, KW), jnp.int32),
        ),
        mesh=_mesh(),
        scratch_types=[
            pltpu.VMEM((BSTAGE, NBY), jnp.float32),
            pltpu.VMEM((2, NBY, KW), jnp.int32),
            pltpu.VMEM((3, CHUNK), jnp.float32),
            pltpu.VMEM((3, CHUNK), jnp.float32),
            pltpu.VMEM((3, CHUNK), jnp.float32),
            pltpu.VMEM((3, CHUNK), jnp.float32),
            pltpu.VMEM((2, CHUNK), jnp.int32),
            pltpu.VMEM((2, CHUNK), jnp.float32),
            pltpu.VMEM((2, CHUNK), jnp.float32),
            pltpu.VMEM((2, CHUNK, KW), jnp.int32),
            pltpu.VMEM((2, CHUNK), jnp.float32),
            pltpu.SemaphoreType.DMA,
            pltpu.SemaphoreType.DMA,
            pltpu.SemaphoreType.DMA,
            pltpu.SemaphoreType.DMA,
            pltpu.SemaphoreType.DMA,
            pltpu.SemaphoreType.DMA,
            pltpu.SemaphoreType.DMA,
        ],
        compiler_params=_params(),
    )(_fused_body)
    area, _ = f(pos, node_size_x, node_size_y, utilization_map)
    return area


# speedup vs baseline: 1.0058x; 1.0058x over previous
import functools

import jax
import jax.numpy as jnp
from jax import lax
from jax.experimental import pallas as pl
from jax.experimental.pallas import tpu as pltpu
from jax.experimental.pallas import tpu_sc as plsc

NBX = 512
NBY = 512
NMOV = 1000000
BSX = 1.0 / NBX
BSY = 1.0 / NBY
K = 4
KW = K * K // 2

NC = 2
NS = 16
NW = NC * NS
L = 16

CHUNK = 1600
NCHUNK = NMOV // CHUNK
ITERS = 24

ROWS_PER_TILE = NBX // NS
BSTAGE = 40


def _mesh():
    return plsc.VectorSubcoreMesh(
        core_axis_name="c", subcore_axis_name="s",
        num_cores=NC, num_subcores=NS)


def _params():
    return pltpu.CompilerParams(
        needs_layout_passes=False, use_tc_tiling_on_sc=False)


def _splat_i32(x):
    return jnp.full((L,), 0, jnp.int32) + x


def _pack_pair(a, b):
    return plsc.bitcast(
        plsc.pack(a, b, format=plsc.PackFormat.INTERLEAVED), jnp.int32)


def _unpack_pair(w):
    lo = plsc.bitcast(lax.shift_left(w, 16), jnp.float32)
    hi = plsc.bitcast(w, jnp.float32)
    return lo, hi


def _fused_body(pos_hbm, nsx_hbm, nsy_hbm, map_hbm, area_hbm, table_hbm,
                rowbuf, obuf, xv, yv, sxv, syv, idxv, lxv, lyv, patches,
                areav, bsem0, bsem1, gsem0, gsem1, isem, osem0, osem1):
    c = lax.axis_index("c")
    sid = lax.axis_index("s")
    wid = sid * NC + c
    iota = lax.broadcasted_iota(jnp.int32, (L,), 0)
    bsems = (bsem0, bsem1)
    gsems = (gsem0, gsem1)
    osems = (osem0, osem1)

    r0 = sid * ROWS_PER_TILE
    base = jnp.minimum(r0, NBX - BSTAGE)
    pltpu.sync_copy(map_hbm.at[pl.ds(base, BSTAGE)], rowbuf)
    tb = c * (NBX * NBY)

    def build_row(r, pr, t):
        @pl.when(t >= 1)
        def _():
            pltpu.make_async_copy(
                obuf.at[pr], table_hbm.at[pl.ds(tb + (r - 2) * NBY, NBY)],
                bsems[pr]).wait()

        ob = obuf.at[pr]
        rlocs = [jnp.minimum(r + kx, NBX - 1) - base for kx in range(K)]

        @plsc.parallel_loop(0, NBY // L - 1, unroll=2)
        def per_cb(cb):
            ci = cb * L + iota
            for kx in range(K):
                vals = [rowbuf[rlocs[kx], pl.ds(cb * L + ky, L)]
                        for ky in range(K)]
                for w in range(2):
                    wv = _pack_pair(vals[2 * w], vals[2 * w + 1])
                    plsc.store_scatter(ob, [ci, _splat_i32(kx * 2 + w)], wv)

        cl = (NBY // L - 1) * L + iota
        for kx in range(K):
            rv = _splat_i32(rlocs[kx])
            vals = []
            for ky in range(K):
                cv = jnp.minimum(cl + ky, NBY - 1)
                vals.append(plsc.load_gather(rowbuf, [rv, cv]))
            for w in range(2):
                wv = _pack_pair(vals[2 * w], vals[2 * w + 1])
                plsc.store_scatter(ob, [cl, _splat_i32(kx * 2 + w)], wv)

        pltpu.async_copy(
            ob, table_hbm.at[pl.ds(tb + r * NBY, NBY)], bsems[pr])

    def build_trip(t, carry):
        build_row(r0 + 2 * t, 0, t)
        build_row(r0 + 2 * t + 1, 1, t)
        return carry

    lax.fori_loop(0, ROWS_PER_TILE // 2, build_trip, 0)

    for rl in (ROWS_PER_TILE - 2, ROWS_PER_TILE - 1):
        pltpu.make_async_copy(
            obuf.at[rl % 2], table_hbm.at[pl.ds(tb + (r0 + rl) * NBY, NBY)],
            bsems[rl % 2]).wait()

    plsc.subcore_barrier()

    def gather_descs(p, make):
        return [make(table_hbm.at[idxv.at[p]], patches.at[p], gsems[p])]

    def input_descs(cid, q, make):
        off = cid * CHUNK
        return [
            make(pos_hbm.at[pl.ds(off, CHUNK)], xv.at[q], isem),
            make(pos_hbm.at[pl.ds(NMOV + off, CHUNK)], yv.at[q], isem),
            make(nsx_hbm.at[pl.ds(off, CHUNK)], sxv.at[q], isem),
            make(nsy_hbm.at[pl.ds(off, CHUNK)], syv.at[q], isem),
        ]

    def fire_inputs(cid, q):
        @pl.when(cid < NCHUNK)
        def _():
            input_descs(cid, q, pltpu.async_copy)

    def prep(cid, p, q):
        @pl.when(cid < NCHUNK)
        def _():
            for d in input_descs(cid, q, pltpu.make_async_copy):
                d.wait()

            @plsc.parallel_loop(0, CHUNK // L, unroll=4)
            def idx_pass(n0):
                s = pl.ds(n0 * L, L)
                ix = (xv[q, s] * float(NBX)).astype(jnp.int32)
                iy = (yv[q, s] * float(NBY)).astype(jnp.int32)
                idxv[p, s] = ix * NBY + iy + tb
                lxv[p, s] = ix.astype(jnp.float32) * BSX + BSX
                lyv[p, s] = iy.astype(jnp.float32) * BSY + BSY

            gather_descs(p, pltpu.async_copy)

    fire_inputs(wid, 0)
    fire_inputs(wid + NW, 1)
    prep(wid, 0, 0)

    def process(i, p, q, j6, drain_always):
        cid = wid + NW * i
        fire_inputs(wid + NW * (i + 2), (q + 2) % 3)
        prep(wid + NW * (i + 1), 1 - p, (q + 1) % 3)

        @pl.when(cid < NCHUNK)
        def _():
            for d in gather_descs(p, pltpu.make_async_copy):
                d.wait()

            def drain():
                pltpu.make_async_copy(
                    areav.at[p], area_hbm.at[pl.ds(cid * CHUNK, CHUNK)],
                    osems[p]).wait()

            if drain_always:
                drain()
            else:
                pl.when(j6 >= 1)(drain)

            up = patches.at[p]

            @plsc.parallel_loop(0, CHUNK // L, unroll=2)
            def red_pass(n0):
                s = pl.ds(n0 * L, L)
                x = xv[q, s]
                y = yv[q, s]
                xmax = jnp.minimum(x + sxv[q, s], 1.0)
                ymax = jnp.minimum(y + syv[q, s], 1.0)
                lx1 = lxv[p, s]
                ly1 = lyv[p, s]
                ax = xmax - lx1
                ay = ymax - ly1
                ovx = [jnp.minimum(xmax, lx1) - x,
                       jnp.maximum(jnp.minimum(ax, BSX), 0.0),
                       jnp.maximum(jnp.minimum(ax - BSX, BSX), 0.0),
                       jnp.maximum(ax - 2 * BSX, 0.0)]
                ovy = [jnp.minimum(ymax, ly1) - y,
                       jnp.maximum(jnp.minimum(ay, BSY), 0.0),
                       jnp.maximum(jnp.minimum(ay - BSY, BSY), 0.0),
                       jnp.maximum(ay - 2 * BSY, 0.0)]
                rowi = _splat_i32(n0 * L) + iota
                acc = jnp.zeros((L,), jnp.float32)
                for kx in range(K):
                    w0 = plsc.load_gather(up, [rowi, _splat_i32(kx * 2)])
                    w1 = plsc.load_gather(up, [rowi, _splat_i32(kx * 2 + 1)])
                    u0, u1 = _unpack_pair(w0)
                    u2, u3 = _unpack_pair(w1)
                    t = ((ovy[0] * u0 + ovy[1] * u1)
                         + (ovy[2] * u2 + ovy[3] * u3))
                    acc = acc + ovx[kx] * t
                areav[p, s] = acc

            pltpu.async_copy(
                areav.at[p], area_hbm.at[pl.ds(cid * CHUNK, CHUNK)],
                osems[p])

    def per_six(j6, carry):
        for k in range(6):
            process(6 * j6 + k, k % 2, k % 3, j6, k >= 2)
        return carry

    lax.fori_loop(0, ITERS // 6, per_six, 0)

    nv = lax.div(NCHUNK - wid + NW - 1, NW)
    for p in range(2):
        i_p = jnp.where(lax.rem(nv - 1, 2) == p, nv - 1, nv - 2)
        lastc = wid + NW * i_p

        @pl.when(i_p >= 0)
        def _(lastc=lastc, p=p):
            pltpu.make_async_copy(
                areav.at[p], area_hbm.at[pl.ds(lastc * CHUNK, CHUNK)],
                osems[p]).wait()


def kernel(pos, node_size_x, node_size_y, utilization_map):
    f = functools.partial(
        pl.kernel,
        out_type=(
            jax.ShapeDtypeStruct((NMOV,), jnp.float32),
            jax.ShapeDtypeStruct((NC * NBX * NBY, KW), jnp.int32),
        ),
        mesh=_mesh(),
        scratch_types=[
            pltpu.VMEM((BSTAGE, NBY), jnp.float32),
            pltpu.VMEM((2, NBY, KW), jnp.int32),
            pltpu.VMEM((3, CHUNK), jnp.float32),
            pltpu.VMEM((3, CHUNK), jnp.float32),
            pltpu.VMEM((3, CHUNK), jnp.float32),
            pltpu.VMEM((3, CHUNK), jnp.float32),
            pltpu.VMEM((2, CHUNK), jnp.int32),
            pltpu.VMEM((2, CHUNK), jnp.float32),
            pltpu.VMEM((2, CHUNK), jnp.float32),
            pltpu.VMEM((2, CHUNK, KW), jnp.int32),
            pltpu.VMEM((2, CHUNK), jnp.float32),
            pltpu.SemaphoreType.DMA,
            pltpu.SemaphoreType.DMA,
            pltpu.SemaphoreType.DMA,
            pltpu.SemaphoreType.DMA,
            pltpu.SemaphoreType.DMA,
            pltpu.SemaphoreType.DMA,
            pltpu.SemaphoreType.DMA,
        ],
        compiler_params=_params(),
    )(_fused_body)
    area, _ = f(pos, node_size_x, node_size_y, utilization_map)
    return area
